# baseline (speedup 1.0000x reference)
import functools

import jax
import jax.numpy as jnp
from jax import lax
from jax.experimental import pallas as pl
from jax.experimental.pallas import tpu as pltpu
from jax.experimental.pallas import tpu_sc as plsc

NUM_EXPERTS = 8
TOP_K = 2
HIDDEN = 2048
INTER = 1792
TOKENS = 4096

TB = 256
B = 128
NB = 64 + 7
NP = NB * B
NP_PAD = 9216
NW = 32
ROWS_PER_W = NP_PAD // NW
TOK_PER_W = TOKENS // NW


def _router_body(x_ref, rw_ref, idx_ref, w_ref):
    x = x_ref[...].astype(jnp.bfloat16)
    rw = rw_ref[...].astype(jnp.bfloat16)
    logits = lax.dot_general(
        x, rw, (((1,), (1,)), ((), ())),
        preferred_element_type=jnp.float32)
    e_iota = lax.broadcasted_iota(jnp.int32, logits.shape, 1)
    m1 = jnp.max(logits, axis=1, keepdims=True)
    i1 = jnp.min(jnp.where(logits == m1, e_iota, NUM_EXPERTS), axis=1,
                 keepdims=True)
    masked = jnp.where(e_iota == i1, -jnp.inf, logits)
    m2 = jnp.max(masked, axis=1, keepdims=True)
    i2 = jnp.min(jnp.where(masked == m2, e_iota, NUM_EXPERTS), axis=1,
                 keepdims=True)
    b = jnp.exp(m2 - m1)
    denom = 1.0 + b
    idx_ref[...] = jnp.concatenate([i1, i2], axis=1)
    w_ref[...] = jnp.concatenate([1.0 / denom, b / denom], axis=1)


def _rank_body(e_ref, pos_ref, blk_ref):
    a = e_ref[...]
    rows, cols = a.shape
    r1 = lax.broadcasted_iota(jnp.int32, (rows, rows), 0)
    c1 = lax.broadcasted_iota(jnp.int32, (rows, rows), 1)
    l_strict = jnp.where(r1 > c1, 1.0, 0.0).astype(jnp.bfloat16)
    r2 = lax.broadcasted_iota(jnp.int32, (cols, cols), 0)
    c2 = lax.broadcasted_iota(jnp.int32, (cols, cols), 1)
    u_strict = jnp.where(r2 < c2, 1.0, 0.0).astype(jnp.bfloat16)

    j_f = lax.broadcasted_iota(jnp.int32, (1, cols), 1).astype(jnp.float32)
    pos_acc = jnp.zeros(a.shape, jnp.float32)
    blk_acc = jnp.zeros((1, cols), jnp.float32)
    g = jnp.float32(0.0)
    cum_nb = jnp.float32(0.0)
    for e in range(NUM_EXPERTS):
        m = (a == e).astype(jnp.float32)
        mb = m.astype(jnp.bfloat16)
        w_in_row = lax.dot_general(
            mb, u_strict, (((1,), (0,)), ((), ())),
            preferred_element_type=jnp.float32)
        q = lax.dot_general(
            l_strict, mb, (((1,), (0,)), ((), ())),
            preferred_element_type=jnp.float32)
        prefix_row = jnp.sum(q, axis=1, keepdims=True)
        rank = prefix_row + w_in_row
        c_e = jnp.sum(m)
        nb_e = jnp.floor((c_e + (B - 1.0)) / B)
        pos_acc = pos_acc + m * (g + rank)
        g = g + nb_e * B
        cum_nb = cum_nb + nb_e
        blk_acc = blk_acc + jnp.where(j_f >= cum_nb, 1.0, 0.0)
    pos_ref[...] = pos_acc.astype(jnp.int32)
    blk_ref[...] = jnp.minimum(blk_acc, NUM_EXPERTS - 1.0).astype(jnp.int32)


def _scatter_body(pos_hbm, src_hbm, pos_v, src_v, zero16):
    wid = lax.axis_index("s") * 2 + lax.axis_index("c")

    @pl.when(wid == 0)
    def _():
        pltpu.sync_copy(pos_hbm, pos_v)

        def zero_step(i, _):
            src_v[pl.ds(i * 16, 16)] = jnp.zeros((16,), jnp.int32)
            return 0

        lax.fori_loop(0, NP_PAD // 16, zero_step, 0)

        def scat_step(i, _):
            pv = pos_v[pl.ds(i * 16, 16)]
            vals = lax.broadcasted_iota(jnp.int32, (16,), 0) + i * 16
            plsc.store_scatter(src_v, [pv], vals)
            return 0

        lax.fori_loop(0, (TOKENS * TOP_K) // 16, scat_step, 0)
        pltpu.sync_copy(src_v, src_hbm)

    del zero16


GC = 24


def _gather_body(src_hbm, x_hbm, xs_hbm, idx0, idx1, rows0, rows1,
                 sem0, sem1):
    wid = lax.axis_index("s") * 2 + lax.axis_index("c")
    base = wid * ROWS_PER_W
    n_chunks = ROWS_PER_W // GC
    idx = (idx0, idx1)
    rows = (rows0, rows1)
    sems = (sem0, sem1)

    def start(j):
        b = j % 2
        pltpu.sync_copy(src_hbm.at[pl.ds(base + j * GC, GC)], idx[b])
        pltpu.async_copy(x_hbm.at[idx[b]], rows[b], sems[b])

    start(0)
    for j in range(n_chunks):
        if j + 1 < n_chunks:
            start(j + 1)
        b = j % 2
        pltpu.make_async_copy(x_hbm.at[idx[b]], rows[b], sems[b]).wait()
        pltpu.sync_copy(rows[b], xs_hbm.at[pl.ds(base + j * GC, GC)])


IB = 2
IQ = INTER // IB


def _grouped_body(blk_ref, x_ref, wg_ref, wu_ref, w2_ref, y_ref, xb_ref):
    del blk_ref
    q = pl.program_id(1)

    @pl.when(q == 0)
    def _():
        xb_ref[...] = x_ref[...].astype(jnp.bfloat16)

    xb = xb_ref[...]
    wg = wg_ref[0].astype(jnp.bfloat16)
    wu = wu_ref[0].astype(jnp.bfloat16)
    gate = lax.dot_general(
        xb, wg, (((1,), (1,)), ((), ())),
        preferred_element_type=jnp.float32)
    up = lax.dot_general(
        xb, wu, (((1,), (1,)), ((), ())),
        preferred_element_type=jnp.float32)
    act = (gate * lax.logistic(gate) * up).astype(jnp.bfloat16)
    w2c = w2_ref[0].astype(jnp.bfloat16)
    o = lax.dot_general(
        act, w2c, (((1,), (1,)), ((), ())),
        preferred_element_type=jnp.float32)

    @pl.when(q == 0)
    def _():
        y_ref[...] = o

    @pl.when(q > 0)
    def _():
        y_ref[...] += o


def _shared_body(x_ref, wg_ref, wu_ref, wd_ref, out_ref, xb_ref):
    h = pl.program_id(1)

    @pl.when(h == 0)
    def _():
        xb_ref[...] = x_ref[...].astype(jnp.bfloat16)

    xb = xb_ref[...]
    wg = wg_ref[0]
    wu = wu_ref[0]
    gate = lax.dot_general(
        xb, wg, (((1,), (1,)), ((), ())),
        preferred_element_type=jnp.float32)
    up = lax.dot_general(
        xb, wu, (((1,), (1,)), ((), ())),
        preferred_element_type=jnp.float32)
    act = (gate * lax.logistic(gate) * up).astype(jnp.bfloat16)
    wd = wd_ref[...]
    o = lax.dot_general(
        act, wd, (((1,), (1,)), ((), ())),
        preferred_element_type=jnp.float32)

    @pl.when(h == 0)
    def _():
        out_ref[...] = o

    @pl.when(h > 0)
    def _():
        out_ref[...] += o


def _combine_body(y_hbm, pos_hbm, w_hbm, sh_hbm, out_hbm,
                  idx0, idx1, w_v, yb0, yb1, shb_v, ob_v, sem0, sem1):
    wid = lax.axis_index("s") * 2 + lax.axis_index("c")
    tok0 = wid * TOK_PER_W
    pair0 = wid * TOK_PER_W * 2
    n_chunks = TOK_PER_W // 8
    pltpu.sync_copy(w_hbm.at[pl.ds(pair0, TOK_PER_W * 2)], w_v)
    idx = (idx0, idx1)
    yb = (yb0, yb1)
    sems = (sem0, sem1)

    def start(c, b):
        pltpu.sync_copy(pos_hbm.at[pl.ds(pair0 + c * 16, 16)], idx[b])
        pltpu.async_copy(y_hbm.at[idx[b]], yb[b], sems[b])

    def compute(c, b):
        pltpu.make_async_copy(y_hbm.at[idx[b]], yb[b], sems[b]).wait()
        pltpu.sync_copy(sh_hbm.at[pl.ds(tok0 + c * 8, 8)], shb_v)
        for r in range(8):
            w0 = plsc.load_gather(
                w_v, [jnp.full((16,), c * 16 + 2 * r, jnp.int32)])
            w1 = plsc.load_gather(
                w_v, [jnp.full((16,), c * 16 + 2 * r + 1, jnp.int32)])

            def dstep(d, _):
                ds = pl.ds(d * 16, 16)
                ob_v[r, ds] = (shb_v[r, ds] + w0 * yb[b][2 * r, ds]
                               + w1 * yb[b][2 * r + 1, ds])
                return 0

            lax.fori_loop(0, HIDDEN // 16, dstep, 0)
        pltpu.sync_copy(ob_v, out_hbm.at[pl.ds(tok0 + c * 8, 8)])

    start(0, 0)

    def c2step(c2, _):
        a = 2 * c2
        start(a + 1, 1)
        compute(a, 0)

        @pl.when(c2 < n_chunks // 2 - 1)
        def _():
            start(a + 2, 0)

        compute(a + 1, 1)
        return 0

    lax.fori_loop(0, n_chunks // 2, c2step, 0)


@jax.jit
def kernel(hidden_states, router_weight, w13, w2, w_gate, w_up, w_down):
    T, H = hidden_states.shape
    E = NUM_EXPERTS
    I = INTER
    n_tb = T // TB

    topk_idx, topk_w = pl.pallas_call(
        _router_body,
        out_shape=(jax.ShapeDtypeStruct((T, 2), jnp.int32),
                   jax.ShapeDtypeStruct((T, 2), jnp.float32)),
        grid=(n_tb,),
        in_specs=[
            pl.BlockSpec((TB, H), lambda t: (t, 0)),
            pl.BlockSpec((E, H), lambda t: (0, 0)),
        ],
        out_specs=(pl.BlockSpec((TB, 2), lambda t: (t, 0)),
                   pl.BlockSpec((TB, 2), lambda t: (t, 0))),
    )(hidden_states, router_weight)

    flat_e = topk_idx.reshape(64, 128)
    pos64, blk = pl.pallas_call(
        _rank_body,
        out_shape=(jax.ShapeDtypeStruct((64, 128), jnp.int32),
                   jax.ShapeDtypeStruct((1, 128), jnp.int32)),
    )(flat_e)
    pos = pos64.reshape(T * 2)
    blk_s = blk.reshape(128)

    src = pl.kernel(
        _scatter_body,
        out_type=jax.ShapeDtypeStruct((NP_PAD,), jnp.int32),
        mesh=plsc.VectorSubcoreMesh(core_axis_name="c", subcore_axis_name="s"),
        compiler_params=pltpu.CompilerParams(needs_layout_passes=False),
        scratch_types=[
            pltpu.VMEM((T * 2,), jnp.int32),
            pltpu.VMEM((NP_PAD,), jnp.int32),
            pltpu.VMEM((16,), jnp.int32),
        ],
    )(pos)

    x_sorted = pl.kernel(
        _gather_body,
        out_type=jax.ShapeDtypeStruct((NP_PAD, H), jnp.float32),
        mesh=plsc.VectorSubcoreMesh(core_axis_name="c", subcore_axis_name="s"),
        compiler_params=pltpu.CompilerParams(needs_layout_passes=False),
        scratch_types=[
            pltpu.VMEM((GC,), jnp.int32),
            pltpu.VMEM((GC,), jnp.int32),
            pltpu.VMEM((GC, H), jnp.float32),
            pltpu.VMEM((GC, H), jnp.float32),
            pltpu.SemaphoreType.DMA,
            pltpu.SemaphoreType.DMA,
        ],
    )(src, hidden_states)

    y = pl.pallas_call(
        _grouped_body,
        out_shape=jax.ShapeDtypeStruct((NP_PAD, H), jnp.float32),
        grid_spec=pltpu.PrefetchScalarGridSpec(
            num_scalar_prefetch=1,
            grid=(NB, IB),
            in_specs=[
                pl.BlockSpec((B, H), lambda i, q, b: (i, 0)),
                pl.BlockSpec((1, IQ, H), lambda i, q, b: (b[i], q, 0)),
                pl.BlockSpec((1, IQ, H), lambda i, q, b: (b[i], IB + q, 0)),
                pl.BlockSpec((1, H, IQ), lambda i, q, b: (b[i], 0, q)),
            ],
            out_specs=pl.BlockSpec((B, H), lambda i, q, b: (i, 0)),
            scratch_shapes=[pltpu.VMEM((B, H), jnp.bfloat16)],
        ),
        compiler_params=pltpu.CompilerParams(
            dimension_semantics=("parallel", "arbitrary")),
    )(blk_s, x_sorted, w13, w13, w2)

    wgb = w_gate.astype(jnp.bfloat16).reshape(2, I, H)
    wub = w_up.astype(jnp.bfloat16).reshape(2, I, H)
    wdb = w_down.astype(jnp.bfloat16)

    shared_out = pl.pallas_call(
        _shared_body,
        out_shape=jax.ShapeDtypeStruct((T, H), jnp.float32),
        grid=(n_tb, 2),
        in_specs=[
            pl.BlockSpec((TB, H), lambda t, h: (t, 0)),
            pl.BlockSpec((1, I, H), lambda t, h: (h, 0, 0)),
            pl.BlockSpec((1, I, H), lambda t, h: (h, 0, 0)),
            pl.BlockSpec((H, I), lambda t, h: (0, h)),
        ],
        out_specs=pl.BlockSpec((TB, H), lambda t, h: (t, 0)),
        scratch_shapes=[pltpu.VMEM((TB, H), jnp.bfloat16)],
        compiler_params=pltpu.CompilerParams(
            dimension_semantics=("parallel", "arbitrary")),
    )(hidden_states, wgb, wub, wdb)

    out = pl.kernel(
        _combine_body,
        out_type=jax.ShapeDtypeStruct((T, H), jnp.float32),
        mesh=plsc.VectorSubcoreMesh(core_axis_name="c", subcore_axis_name="s"),
        compiler_params=pltpu.CompilerParams(needs_layout_passes=False),
        scratch_types=[
            pltpu.VMEM((16,), jnp.int32),
            pltpu.VMEM((16,), jnp.int32),
            pltpu.VMEM((TOK_PER_W * 2,), jnp.float32),
            pltpu.VMEM((16, H), jnp.float32),
            pltpu.VMEM((16, H), jnp.float32),
            pltpu.VMEM((8, H), jnp.float32),
            pltpu.VMEM((8, H), jnp.float32),
            pltpu.SemaphoreType.DMA,
            pltpu.SemaphoreType.DMA,
        ],
    )(y, pos, topk_w.reshape(T * 2), shared_out)

    return out


# speedup vs baseline: 1.2054x; 1.2054x over previous
import functools

import jax
import jax.numpy as jnp
from jax import lax
from jax.experimental import pallas as pl
from jax.experimental.pallas import tpu as pltpu
from jax.experimental.pallas import tpu_sc as plsc

NUM_EXPERTS = 8
TOP_K = 2
HIDDEN = 2048
INTER = 1792
TOKENS = 4096

TB = 256
B = 128
NB = 64 + 7
NP = NB * B
NP_PAD = 9216
NW = 32
ROWS_PER_W = NP_PAD // NW
TOK_PER_W = TOKENS // NW


def _router_body(x_ref, rw_ref, idx_ref, w_ref):
    x = x_ref[...].astype(jnp.bfloat16)
    rw = rw_ref[...].astype(jnp.bfloat16)
    logits = lax.dot_general(
        x, rw, (((1,), (1,)), ((), ())),
        preferred_element_type=jnp.float32)
    e_iota = lax.broadcasted_iota(jnp.int32, logits.shape, 1)
    m1 = jnp.max(logits, axis=1, keepdims=True)
    i1 = jnp.min(jnp.where(logits == m1, e_iota, NUM_EXPERTS), axis=1,
                 keepdims=True)
    masked = jnp.where(e_iota == i1, -jnp.inf, logits)
    m2 = jnp.max(masked, axis=1, keepdims=True)
    i2 = jnp.min(jnp.where(masked == m2, e_iota, NUM_EXPERTS), axis=1,
                 keepdims=True)
    b = jnp.exp(m2 - m1)
    denom = 1.0 + b
    idx_ref[...] = jnp.concatenate([i1, i2], axis=1)
    w_ref[...] = jnp.concatenate([1.0 / denom, b / denom], axis=1)


def _rank_body(e_ref, pos_ref, blk_ref):
    a = e_ref[...]
    rows, cols = a.shape
    r1 = lax.broadcasted_iota(jnp.int32, (rows, rows), 0)
    c1 = lax.broadcasted_iota(jnp.int32, (rows, rows), 1)
    l_strict = jnp.where(r1 > c1, 1.0, 0.0).astype(jnp.bfloat16)
    r2 = lax.broadcasted_iota(jnp.int32, (cols, cols), 0)
    c2 = lax.broadcasted_iota(jnp.int32, (cols, cols), 1)
    u_strict = jnp.where(r2 < c2, 1.0, 0.0).astype(jnp.bfloat16)

    j_f = lax.broadcasted_iota(jnp.int32, (1, cols), 1).astype(jnp.float32)
    pos_acc = jnp.zeros(a.shape, jnp.float32)
    blk_acc = jnp.zeros((1, cols), jnp.float32)
    g = jnp.float32(0.0)
    cum_nb = jnp.float32(0.0)
    for e in range(NUM_EXPERTS):
        m = (a == e).astype(jnp.float32)
        mb = m.astype(jnp.bfloat16)
        w_in_row = lax.dot_general(
            mb, u_strict, (((1,), (0,)), ((), ())),
            preferred_element_type=jnp.float32)
        q = lax.dot_general(
            l_strict, mb, (((1,), (0,)), ((), ())),
            preferred_element_type=jnp.float32)
        prefix_row = jnp.sum(q, axis=1, keepdims=True)
        rank = prefix_row + w_in_row
        c_e = jnp.sum(m)
        nb_e = jnp.floor((c_e + (B - 1.0)) / B)
        pos_acc = pos_acc + m * (g + rank)
        g = g + nb_e * B
        cum_nb = cum_nb + nb_e
        blk_acc = blk_acc + jnp.where(j_f >= cum_nb, 1.0, 0.0)
    pos_ref[...] = pos_acc.astype(jnp.int32)
    blk_ref[...] = jnp.minimum(blk_acc, NUM_EXPERTS - 1.0).astype(jnp.int32)


def _scatter_body(pos_hbm, src_hbm, pos_v, src_v, zero16):
    wid = lax.axis_index("s") * 2 + lax.axis_index("c")

    @pl.when(wid == 0)
    def _():
        pltpu.sync_copy(pos_hbm, pos_v)

        def zero_step(i, _):
            src_v[pl.ds(i * 16, 16)] = jnp.zeros((16,), jnp.int32)
            return 0

        lax.fori_loop(0, NP_PAD // 16, zero_step, 0)

        def scat_step(i, _):
            pv = pos_v[pl.ds(i * 16, 16)]
            vals = lax.broadcasted_iota(jnp.int32, (16,), 0) + i * 16
            plsc.store_scatter(src_v, [pv], vals)
            return 0

        lax.fori_loop(0, (TOKENS * TOP_K) // 16, scat_step, 0)
        pltpu.sync_copy(src_v, src_hbm)

    del zero16


GC = 24


def _gather_body(src_hbm, x_hbm, xs_hbm, idx0, idx1, rows0, rows1,
                 sem0, sem1):
    wid = lax.axis_index("s") * 2 + lax.axis_index("c")
    base = wid * ROWS_PER_W
    n_chunks = ROWS_PER_W // GC
    idx = (idx0, idx1)
    rows = (rows0, rows1)
    sems = (sem0, sem1)

    def start(j):
        b = j % 2
        pltpu.sync_copy(src_hbm.at[pl.ds(base + j * GC, GC)], idx[b])
        pltpu.async_copy(x_hbm.at[idx[b]], rows[b], sems[b])

    start(0)
    for j in range(n_chunks):
        if j + 1 < n_chunks:
            start(j + 1)
        b = j % 2
        pltpu.make_async_copy(x_hbm.at[idx[b]], rows[b], sems[b]).wait()
        pltpu.sync_copy(rows[b], xs_hbm.at[pl.ds(base + j * GC, GC)])


def _grouped_body(blk_ref, x_ref, w13_ref, w2_ref, y_ref):
    del blk_ref
    xb = x_ref[...].astype(jnp.bfloat16)
    w13 = w13_ref[0]
    gu = lax.dot_general(
        xb, w13, (((1,), (1,)), ((), ())),
        preferred_element_type=jnp.float32)
    gate = gu[:, :INTER]
    up = gu[:, INTER:]
    act = (gate * lax.logistic(gate) * up).astype(jnp.bfloat16)
    w2 = w2_ref[0]
    y_ref[...] = lax.dot_general(
        act, w2, (((1,), (1,)), ((), ())),
        preferred_element_type=jnp.float32)


def _shared_body(x_ref, wg_ref, wu_ref, wd_ref, out_ref, xb_ref):
    h = pl.program_id(1)

    @pl.when(h == 0)
    def _():
        xb_ref[...] = x_ref[...].astype(jnp.bfloat16)

    xb = xb_ref[...]
    wg = wg_ref[0]
    wu = wu_ref[0]
    gate = lax.dot_general(
        xb, wg, (((1,), (1,)), ((), ())),
        preferred_element_type=jnp.float32)
    up = lax.dot_general(
        xb, wu, (((1,), (1,)), ((), ())),
        preferred_element_type=jnp.float32)
    act = (gate * lax.logistic(gate) * up).astype(jnp.bfloat16)
    wd = wd_ref[...]
    o = lax.dot_general(
        act, wd, (((1,), (1,)), ((), ())),
        preferred_element_type=jnp.float32)

    @pl.when(h == 0)
    def _():
        out_ref[...] = o

    @pl.when(h > 0)
    def _():
        out_ref[...] += o


def _combine_body(y_hbm, pos_hbm, w_hbm, sh_hbm, out_hbm,
                  idx0, idx1, w_v, yb0, yb1, shb_v, ob_v, sem0, sem1):
    wid = lax.axis_index("s") * 2 + lax.axis_index("c")
    tok0 = wid * TOK_PER_W
    pair0 = wid * TOK_PER_W * 2
    n_chunks = TOK_PER_W // 8
    pltpu.sync_copy(w_hbm.at[pl.ds(pair0, TOK_PER_W * 2)], w_v)
    idx = (idx0, idx1)
    yb = (yb0, yb1)
    sems = (sem0, sem1)

    def start(c, b):
        pltpu.sync_copy(pos_hbm.at[pl.ds(pair0 + c * 16, 16)], idx[b])
        pltpu.async_copy(y_hbm.at[idx[b]], yb[b], sems[b])

    def compute(c, b):
        pltpu.make_async_copy(y_hbm.at[idx[b]], yb[b], sems[b]).wait()
        pltpu.sync_copy(sh_hbm.at[pl.ds(tok0 + c * 8, 8)], shb_v)
        for r in range(8):
            w0 = plsc.load_gather(
                w_v, [jnp.full((16,), c * 16 + 2 * r, jnp.int32)])
            w1 = plsc.load_gather(
                w_v, [jnp.full((16,), c * 16 + 2 * r + 1, jnp.int32)])

            def dstep(d, _):
                ds = pl.ds(d * 16, 16)
                ob_v[r, ds] = (shb_v[r, ds] + w0 * yb[b][2 * r, ds]
                               + w1 * yb[b][2 * r + 1, ds])
                return 0

            lax.fori_loop(0, HIDDEN // 16, dstep, 0)
        pltpu.sync_copy(ob_v, out_hbm.at[pl.ds(tok0 + c * 8, 8)])

    start(0, 0)

    def c2step(c2, _):
        a = 2 * c2
        start(a + 1, 1)
        compute(a, 0)

        @pl.when(c2 < n_chunks // 2 - 1)
        def _():
            start(a + 2, 0)

        compute(a + 1, 1)
        return 0

    lax.fori_loop(0, n_chunks // 2, c2step, 0)


@jax.jit
def kernel(hidden_states, router_weight, w13, w2, w_gate, w_up, w_down):
    T, H = hidden_states.shape
    E = NUM_EXPERTS
    I = INTER
    n_tb = T // TB

    topk_idx, topk_w = pl.pallas_call(
        _router_body,
        out_shape=(jax.ShapeDtypeStruct((T, 2), jnp.int32),
                   jax.ShapeDtypeStruct((T, 2), jnp.float32)),
        grid=(n_tb,),
        in_specs=[
            pl.BlockSpec((TB, H), lambda t: (t, 0)),
            pl.BlockSpec((E, H), lambda t: (0, 0)),
        ],
        out_specs=(pl.BlockSpec((TB, 2), lambda t: (t, 0)),
                   pl.BlockSpec((TB, 2), lambda t: (t, 0))),
    )(hidden_states, router_weight)

    flat_e = topk_idx.reshape(64, 128)
    pos64, blk = pl.pallas_call(
        _rank_body,
        out_shape=(jax.ShapeDtypeStruct((64, 128), jnp.int32),
                   jax.ShapeDtypeStruct((1, 128), jnp.int32)),
    )(flat_e)
    pos = pos64.reshape(T * 2)
    blk_s = blk.reshape(128)

    src = pl.kernel(
        _scatter_body,
        out_type=jax.ShapeDtypeStruct((NP_PAD,), jnp.int32),
        mesh=plsc.VectorSubcoreMesh(core_axis_name="c", subcore_axis_name="s"),
        compiler_params=pltpu.CompilerParams(needs_layout_passes=False),
        scratch_types=[
            pltpu.VMEM((T * 2,), jnp.int32),
            pltpu.VMEM((NP_PAD,), jnp.int32),
            pltpu.VMEM((16,), jnp.int32),
        ],
    )(pos)

    x_sorted = pl.kernel(
        _gather_body,
        out_type=jax.ShapeDtypeStruct((NP_PAD, H), jnp.float32),
        mesh=plsc.VectorSubcoreMesh(core_axis_name="c", subcore_axis_name="s"),
        compiler_params=pltpu.CompilerParams(needs_layout_passes=False),
        scratch_types=[
            pltpu.VMEM((GC,), jnp.int32),
            pltpu.VMEM((GC,), jnp.int32),
            pltpu.VMEM((GC, H), jnp.float32),
            pltpu.VMEM((GC, H), jnp.float32),
            pltpu.SemaphoreType.DMA,
            pltpu.SemaphoreType.DMA,
        ],
    )(src, hidden_states)

    w13b = w13.astype(jnp.bfloat16)
    w2b = w2.astype(jnp.bfloat16)

    y = pl.pallas_call(
        _grouped_body,
        out_shape=jax.ShapeDtypeStruct((NP_PAD, H), jnp.float32),
        grid_spec=pltpu.PrefetchScalarGridSpec(
            num_scalar_prefetch=1,
            grid=(NB,),
            in_specs=[
                pl.BlockSpec((B, H), lambda i, b: (i, 0)),
                pl.BlockSpec((1, 2 * I, H), lambda i, b: (b[i], 0, 0)),
                pl.BlockSpec((1, H, I), lambda i, b: (b[i], 0, 0)),
            ],
            out_specs=pl.BlockSpec((B, H), lambda i, b: (i, 0)),
        ),
        compiler_params=pltpu.CompilerParams(
            dimension_semantics=("parallel",)),
    )(blk_s, x_sorted, w13b, w2b)

    wgb = w_gate.astype(jnp.bfloat16).reshape(2, I, H)
    wub = w_up.astype(jnp.bfloat16).reshape(2, I, H)
    wdb = w_down.astype(jnp.bfloat16)

    shared_out = pl.pallas_call(
        _shared_body,
        out_shape=jax.ShapeDtypeStruct((T, H), jnp.float32),
        grid=(n_tb, 2),
        in_specs=[
            pl.BlockSpec((TB, H), lambda t, h: (t, 0)),
            pl.BlockSpec((1, I, H), lambda t, h: (h, 0, 0)),
            pl.BlockSpec((1, I, H), lambda t, h: (h, 0, 0)),
            pl.BlockSpec((H, I), lambda t, h: (0, h)),
        ],
        out_specs=pl.BlockSpec((TB, H), lambda t, h: (t, 0)),
        scratch_shapes=[pltpu.VMEM((TB, H), jnp.bfloat16)],
        compiler_params=pltpu.CompilerParams(
            dimension_semantics=("parallel", "arbitrary")),
    )(hidden_states, wgb, wub, wdb)

    out = pl.kernel(
        _combine_body,
        out_type=jax.ShapeDtypeStruct((T, H), jnp.float32),
        mesh=plsc.VectorSubcoreMesh(core_axis_name="c", subcore_axis_name="s"),
        compiler_params=pltpu.CompilerParams(needs_layout_passes=False),
        scratch_types=[
            pltpu.VMEM((16,), jnp.int32),
            pltpu.VMEM((16,), jnp.int32),
            pltpu.VMEM((TOK_PER_W * 2,), jnp.float32),
            pltpu.VMEM((16, H), jnp.float32),
            pltpu.VMEM((16, H), jnp.float32),
            pltpu.VMEM((8, H), jnp.float32),
            pltpu.VMEM((8, H), jnp.float32),
            pltpu.SemaphoreType.DMA,
            pltpu.SemaphoreType.DMA,
        ],
    )(y, pos, topk_w.reshape(T * 2), shared_out)

    return out


# speedup vs baseline: 1.3498x; 1.1198x over previous
import functools

import jax
import jax.numpy as jnp
from jax import lax
from jax.experimental import pallas as pl
from jax.experimental.pallas import tpu as pltpu
from jax.experimental.pallas import tpu_sc as plsc

NUM_EXPERTS = 8
TOP_K = 2
HIDDEN = 2048
INTER = 1792
TOKENS = 4096

TB = 256
B = 256
NB = 32 + 7
NP = NB * B
NP_PAD = NP
NW = 32
ROWS_PER_W = NP_PAD // NW
TOK_PER_W = TOKENS // NW


def _router_body(x_ref, rw_ref, idx_ref, w_ref):
    x = x_ref[...].astype(jnp.bfloat16)
    rw = rw_ref[...].astype(jnp.bfloat16)
    logits = lax.dot_general(
        x, rw, (((1,), (1,)), ((), ())),
        preferred_element_type=jnp.float32)
    e_iota = lax.broadcasted_iota(jnp.int32, logits.shape, 1)
    m1 = jnp.max(logits, axis=1, keepdims=True)
    i1 = jnp.min(jnp.where(logits == m1, e_iota, NUM_EXPERTS), axis=1,
                 keepdims=True)
    masked = jnp.where(e_iota == i1, -jnp.inf, logits)
    m2 = jnp.max(masked, axis=1, keepdims=True)
    i2 = jnp.min(jnp.where(masked == m2, e_iota, NUM_EXPERTS), axis=1,
                 keepdims=True)
    b = jnp.exp(m2 - m1)
    denom = 1.0 + b
    idx_ref[...] = jnp.concatenate([i1, i2], axis=1)
    w_ref[...] = jnp.concatenate([1.0 / denom, b / denom], axis=1)


def _rank_body(e_ref, pos_ref, blk_ref):
    a = e_ref[...]
    rows, cols = a.shape
    r1 = lax.broadcasted_iota(jnp.int32, (rows, rows), 0)
    c1 = lax.broadcasted_iota(jnp.int32, (rows, rows), 1)
    l_strict = jnp.where(r1 > c1, 1.0, 0.0).astype(jnp.bfloat16)
    r2 = lax.broadcasted_iota(jnp.int32, (cols, cols), 0)
    c2 = lax.broadcasted_iota(jnp.int32, (cols, cols), 1)
    u_strict = jnp.where(r2 < c2, 1.0, 0.0).astype(jnp.bfloat16)

    j_f = lax.broadcasted_iota(jnp.int32, (1, cols), 1).astype(jnp.float32)
    pos_acc = jnp.zeros(a.shape, jnp.float32)
    blk_acc = jnp.zeros((1, cols), jnp.float32)
    g = jnp.float32(0.0)
    cum_nb = jnp.float32(0.0)
    for e in range(NUM_EXPERTS):
        m = (a == e).astype(jnp.float32)
        mb = m.astype(jnp.bfloat16)
        w_in_row = lax.dot_general(
            mb, u_strict, (((1,), (0,)), ((), ())),
            preferred_element_type=jnp.float32)
        q = lax.dot_general(
            l_strict, mb, (((1,), (0,)), ((), ())),
            preferred_element_type=jnp.float32)
        prefix_row = jnp.sum(q, axis=1, keepdims=True)
        rank = prefix_row + w_in_row
        c_e = jnp.sum(m)
        nb_e = jnp.floor((c_e + (B - 1.0)) / B)
        pos_acc = pos_acc + m * (g + rank)
        g = g + nb_e * B
        cum_nb = cum_nb + nb_e
        blk_acc = blk_acc + jnp.where(j_f >= cum_nb, 1.0, 0.0)
    pos_ref[...] = pos_acc.astype(jnp.int32)
    blk_ref[...] = jnp.minimum(blk_acc, NUM_EXPERTS - 1.0).astype(jnp.int32)


def _scatter_body(pos_hbm, src_hbm, pos_v, src_v, zero16):
    wid = lax.axis_index("s") * 2 + lax.axis_index("c")

    @pl.when(wid == 0)
    def _():
        pltpu.sync_copy(pos_hbm, pos_v)

        def zero_step(i, _):
            src_v[pl.ds(i * 16, 16)] = jnp.zeros((16,), jnp.int32)
            return 0

        lax.fori_loop(0, NP_PAD // 16, zero_step, 0)

        def scat_step(i, _):
            pv = pos_v[pl.ds(i * 16, 16)]
            vals = lax.broadcasted_iota(jnp.int32, (16,), 0) + i * 16
            plsc.store_scatter(src_v, [pv], vals)
            return 0

        lax.fori_loop(0, (TOKENS * TOP_K) // 16, scat_step, 0)
        pltpu.sync_copy(src_v, src_hbm)

    del zero16


GC = 24


def _gather_body(src_hbm, x_hbm, xs_hbm, idx0, idx1, rows0, rows1,
                 sem0, sem1):
    wid = lax.axis_index("s") * 2 + lax.axis_index("c")
    base = wid * ROWS_PER_W
    n_chunks = ROWS_PER_W // GC
    idx = (idx0, idx1)
    rows = (rows0, rows1)
    sems = (sem0, sem1)

    def start(j):
        b = j % 2
        pltpu.sync_copy(src_hbm.at[pl.ds(base + j * GC, GC)], idx[b])
        pltpu.async_copy(x_hbm.at[idx[b]], rows[b], sems[b])

    start(0)
    for j in range(n_chunks):
        if j + 1 < n_chunks:
            start(j + 1)
        b = j % 2
        pltpu.make_async_copy(x_hbm.at[idx[b]], rows[b], sems[b]).wait()
        pltpu.sync_copy(rows[b], xs_hbm.at[pl.ds(base + j * GC, GC)])


IB = 4
IQ = INTER // IB


def _grouped_body(blk_ref, x_ref, wg_ref, wu_ref, w2_ref, y_ref, xb_ref):
    del blk_ref
    i = pl.program_id(0)
    q = pl.program_id(1)
    qe = q + (i % 2) * (IB - 1 - 2 * q)

    @pl.when(q == 0)
    def _():
        xb_ref[...] = x_ref[...].astype(jnp.bfloat16)

    xb = xb_ref[...]
    wg = wg_ref[0].astype(jnp.bfloat16)
    wu = wu_ref[0].astype(jnp.bfloat16)
    gate = lax.dot_general(
        xb, wg, (((1,), (1,)), ((), ())),
        preferred_element_type=jnp.float32)
    up = lax.dot_general(
        xb, wu, (((1,), (1,)), ((), ())),
        preferred_element_type=jnp.float32)
    act = (gate * lax.logistic(gate) * up).astype(jnp.bfloat16)
    for k in range(IB):
        @pl.when((qe == k) & (q == 0))
        def _(k=k):
            w2k = w2_ref[0, :, pl.ds(k * IQ, IQ)]
            y_ref[...] = lax.dot_general(
                act, w2k, (((1,), (1,)), ((), ())),
                preferred_element_type=jnp.float32)

        @pl.when((qe == k) & (q > 0))
        def _(k=k):
            w2k = w2_ref[0, :, pl.ds(k * IQ, IQ)]
            y_ref[...] += lax.dot_general(
                act, w2k, (((1,), (1,)), ((), ())),
                preferred_element_type=jnp.float32)


def _shared_body(x_ref, wg_ref, wu_ref, wd_ref, out_ref, xb_ref):
    h = pl.program_id(1)

    @pl.when(h == 0)
    def _():
        xb_ref[...] = x_ref[...].astype(jnp.bfloat16)

    xb = xb_ref[...]
    wg = wg_ref[0]
    wu = wu_ref[0]
    gate = lax.dot_general(
        xb, wg, (((1,), (1,)), ((), ())),
        preferred_element_type=jnp.float32)
    up = lax.dot_general(
        xb, wu, (((1,), (1,)), ((), ())),
        preferred_element_type=jnp.float32)
    act = (gate * lax.logistic(gate) * up).astype(jnp.bfloat16)
    wd = wd_ref[...]
    o = lax.dot_general(
        act, wd, (((1,), (1,)), ((), ())),
        preferred_element_type=jnp.float32)

    @pl.when(h == 0)
    def _():
        out_ref[...] = o

    @pl.when(h > 0)
    def _():
        out_ref[...] += o


def _combine_body(y_hbm, pos_hbm, w_hbm, sh_hbm, out_hbm,
                  idx0, idx1, w_v, yb0, yb1, shb_v, ob_v, sem0, sem1):
    wid = lax.axis_index("s") * 2 + lax.axis_index("c")
    tok0 = wid * TOK_PER_W
    pair0 = wid * TOK_PER_W * 2
    n_chunks = TOK_PER_W // 8
    pltpu.sync_copy(w_hbm.at[pl.ds(pair0, TOK_PER_W * 2)], w_v)
    idx = (idx0, idx1)
    yb = (yb0, yb1)
    sems = (sem0, sem1)

    def start(c, b):
        pltpu.sync_copy(pos_hbm.at[pl.ds(pair0 + c * 16, 16)], idx[b])
        pltpu.async_copy(y_hbm.at[idx[b]], yb[b], sems[b])

    def compute(c, b):
        pltpu.make_async_copy(y_hbm.at[idx[b]], yb[b], sems[b]).wait()
        pltpu.sync_copy(sh_hbm.at[pl.ds(tok0 + c * 8, 8)], shb_v)
        for r in range(8):
            w0 = plsc.load_gather(
                w_v, [jnp.full((16,), c * 16 + 2 * r, jnp.int32)])
            w1 = plsc.load_gather(
                w_v, [jnp.full((16,), c * 16 + 2 * r + 1, jnp.int32)])

            def dstep(d, _):
                ds = pl.ds(d * 16, 16)
                ob_v[r, ds] = (shb_v[r, ds] + w0 * yb[b][2 * r, ds]
                               + w1 * yb[b][2 * r + 1, ds])
                return 0

            lax.fori_loop(0, HIDDEN // 16, dstep, 0)
        pltpu.sync_copy(ob_v, out_hbm.at[pl.ds(tok0 + c * 8, 8)])

    start(0, 0)

    def c2step(c2, _):
        a = 2 * c2
        start(a + 1, 1)
        compute(a, 0)

        @pl.when(c2 < n_chunks // 2 - 1)
        def _():
            start(a + 2, 0)

        compute(a + 1, 1)
        return 0

    lax.fori_loop(0, n_chunks // 2, c2step, 0)


@jax.jit
def kernel(hidden_states, router_weight, w13, w2, w_gate, w_up, w_down):
    T, H = hidden_states.shape
    E = NUM_EXPERTS
    I = INTER
    n_tb = T // TB

    topk_idx, topk_w = pl.pallas_call(
        _router_body,
        out_shape=(jax.ShapeDtypeStruct((T, 2), jnp.int32),
                   jax.ShapeDtypeStruct((T, 2), jnp.float32)),
        grid=(n_tb,),
        in_specs=[
            pl.BlockSpec((TB, H), lambda t: (t, 0)),
            pl.BlockSpec((E, H), lambda t: (0, 0)),
        ],
        out_specs=(pl.BlockSpec((TB, 2), lambda t: (t, 0)),
                   pl.BlockSpec((TB, 2), lambda t: (t, 0))),
    )(hidden_states, router_weight)

    flat_e = topk_idx.reshape(64, 128)
    pos64, blk = pl.pallas_call(
        _rank_body,
        out_shape=(jax.ShapeDtypeStruct((64, 128), jnp.int32),
                   jax.ShapeDtypeStruct((1, 128), jnp.int32)),
    )(flat_e)
    pos = pos64.reshape(T * 2)
    blk_s = blk.reshape(128)

    src = pl.kernel(
        _scatter_body,
        out_type=jax.ShapeDtypeStruct((NP_PAD,), jnp.int32),
        mesh=plsc.VectorSubcoreMesh(core_axis_name="c", subcore_axis_name="s"),
        compiler_params=pltpu.CompilerParams(needs_layout_passes=False),
        scratch_types=[
            pltpu.VMEM((T * 2,), jnp.int32),
            pltpu.VMEM((NP_PAD,), jnp.int32),
            pltpu.VMEM((16,), jnp.int32),
        ],
    )(pos)

    x_sorted = pl.kernel(
        _gather_body,
        out_type=jax.ShapeDtypeStruct((NP_PAD, H), jnp.float32),
        mesh=plsc.VectorSubcoreMesh(core_axis_name="c", subcore_axis_name="s"),
        compiler_params=pltpu.CompilerParams(needs_layout_passes=False),
        scratch_types=[
            pltpu.VMEM((GC,), jnp.int32),
            pltpu.VMEM((GC,), jnp.int32),
            pltpu.VMEM((GC, H), jnp.float32),
            pltpu.VMEM((GC, H), jnp.float32),
            pltpu.SemaphoreType.DMA,
            pltpu.SemaphoreType.DMA,
        ],
    )(src, hidden_states)

    w2b = w2.astype(jnp.bfloat16)

    def _zig(i, q):
        return q + (i % 2) * (IB - 1 - 2 * q)

    y = pl.pallas_call(
        _grouped_body,
        out_shape=jax.ShapeDtypeStruct((NP_PAD, H), jnp.float32),
        grid_spec=pltpu.PrefetchScalarGridSpec(
            num_scalar_prefetch=1,
            grid=(NB, IB),
            in_specs=[
                pl.BlockSpec((B, H), lambda i, q, b: (i, 0)),
                pl.BlockSpec((1, IQ, H),
                             lambda i, q, b: (b[i], _zig(i, q), 0)),
                pl.BlockSpec((1, IQ, H),
                             lambda i, q, b: (b[i], IB + _zig(i, q), 0)),
                pl.BlockSpec((1, H, I), lambda i, q, b: (b[i], 0, 0)),
            ],
            out_specs=pl.BlockSpec((B, H), lambda i, q, b: (i, 0)),
            scratch_shapes=[pltpu.VMEM((B, H), jnp.bfloat16)],
        ),
        compiler_params=pltpu.CompilerParams(
            dimension_semantics=("arbitrary", "arbitrary")),
    )(blk_s, x_sorted, w13, w13, w2b)

    wgb = w_gate.astype(jnp.bfloat16).reshape(2, I, H)
    wub = w_up.astype(jnp.bfloat16).reshape(2, I, H)
    wdb = w_down.astype(jnp.bfloat16)

    shared_out = pl.pallas_call(
        _shared_body,
        out_shape=jax.ShapeDtypeStruct((T, H), jnp.float32),
        grid=(n_tb, 2),
        in_specs=[
            pl.BlockSpec((TB, H), lambda t, h: (t, 0)),
            pl.BlockSpec((1, I, H), lambda t, h: (h, 0, 0)),
            pl.BlockSpec((1, I, H), lambda t, h: (h, 0, 0)),
            pl.BlockSpec((H, I), lambda t, h: (0, h)),
        ],
        out_specs=pl.BlockSpec((TB, H), lambda t, h: (t, 0)),
        scratch_shapes=[pltpu.VMEM((TB, H), jnp.bfloat16)],
        compiler_params=pltpu.CompilerParams(
            dimension_semantics=("parallel", "arbitrary")),
    )(hidden_states, wgb, wub, wdb)

    out = pl.kernel(
        _combine_body,
        out_type=jax.ShapeDtypeStruct((T, H), jnp.float32),
        mesh=plsc.VectorSubcoreMesh(core_axis_name="c", subcore_axis_name="s"),
        compiler_params=pltpu.CompilerParams(needs_layout_passes=False),
        scratch_types=[
            pltpu.VMEM((16,), jnp.int32),
            pltpu.VMEM((16,), jnp.int32),
            pltpu.VMEM((TOK_PER_W * 2,), jnp.float32),
            pltpu.VMEM((16, H), jnp.float32),
            pltpu.VMEM((16, H), jnp.float32),
            pltpu.VMEM((8, H), jnp.float32),
            pltpu.VMEM((8, H), jnp.float32),
            pltpu.SemaphoreType.DMA,
            pltpu.SemaphoreType.DMA,
        ],
    )(y, pos, topk_w.reshape(T * 2), shared_out)

    return out


# speedup vs baseline: 1.4312x; 1.0603x over previous
import functools

import jax
import jax.numpy as jnp
from jax import lax
from jax.experimental import pallas as pl
from jax.experimental.pallas import tpu as pltpu
from jax.experimental.pallas import tpu_sc as plsc

NUM_EXPERTS = 8
TOP_K = 2
HIDDEN = 2048
INTER = 1792
TOKENS = 4096

TB = 256
B = 256
NB = 32 + 7
NP = NB * B
NP_PAD = NP
NW = 32
ROWS_PER_W = NP_PAD // NW
TOK_PER_W = TOKENS // NW


def _router_body(x_ref, rw_ref, idx_ref, w_ref):
    x = x_ref[...].astype(jnp.bfloat16)
    rw = rw_ref[...].astype(jnp.bfloat16)
    logits = lax.dot_general(
        x, rw, (((1,), (1,)), ((), ())),
        preferred_element_type=jnp.float32)
    e_iota = lax.broadcasted_iota(jnp.int32, logits.shape, 1)
    m1 = jnp.max(logits, axis=1, keepdims=True)
    i1 = jnp.min(jnp.where(logits == m1, e_iota, NUM_EXPERTS), axis=1,
                 keepdims=True)
    masked = jnp.where(e_iota == i1, -jnp.inf, logits)
    m2 = jnp.max(masked, axis=1, keepdims=True)
    i2 = jnp.min(jnp.where(masked == m2, e_iota, NUM_EXPERTS), axis=1,
                 keepdims=True)
    b = jnp.exp(m2 - m1)
    denom = 1.0 + b
    idx_ref[...] = jnp.concatenate([i1, i2], axis=1)
    w_ref[...] = jnp.concatenate([1.0 / denom, b / denom], axis=1)


def _rank_body(e_ref, pos_ref, blk_ref):
    a = e_ref[...]
    rows, cols = a.shape
    r1 = lax.broadcasted_iota(jnp.int32, (rows, rows), 0)
    c1 = lax.broadcasted_iota(jnp.int32, (rows, rows), 1)
    l_strict = jnp.where(r1 > c1, 1.0, 0.0).astype(jnp.bfloat16)
    r2 = lax.broadcasted_iota(jnp.int32, (cols, cols), 0)
    c2 = lax.broadcasted_iota(jnp.int32, (cols, cols), 1)
    u_strict = jnp.where(r2 < c2, 1.0, 0.0).astype(jnp.bfloat16)

    j_f = lax.broadcasted_iota(jnp.int32, (1, cols), 1).astype(jnp.float32)
    pos_acc = jnp.zeros(a.shape, jnp.float32)
    blk_acc = jnp.zeros((1, cols), jnp.float32)
    g = jnp.float32(0.0)
    cum_nb = jnp.float32(0.0)
    for e in range(NUM_EXPERTS):
        m = (a == e).astype(jnp.float32)
        mb = m.astype(jnp.bfloat16)
        w_in_row = lax.dot_general(
            mb, u_strict, (((1,), (0,)), ((), ())),
            preferred_element_type=jnp.float32)
        q = lax.dot_general(
            l_strict, mb, (((1,), (0,)), ((), ())),
            preferred_element_type=jnp.float32)
        prefix_row = jnp.sum(q, axis=1, keepdims=True)
        rank = prefix_row + w_in_row
        c_e = jnp.sum(m)
        nb_e = jnp.floor((c_e + (B - 1.0)) / B)
        pos_acc = pos_acc + m * (g + rank)
        g = g + nb_e * B
        cum_nb = cum_nb + nb_e
        blk_acc = blk_acc + jnp.where(j_f >= cum_nb, 1.0, 0.0)
    pos_ref[...] = pos_acc.astype(jnp.int32)
    blk_ref[...] = jnp.minimum(blk_acc, NUM_EXPERTS - 1.0).astype(jnp.int32)


def _scatter_body(pos_hbm, src_hbm, pos_v, src_v, zero16):
    wid = lax.axis_index("s") * 2 + lax.axis_index("c")

    @pl.when(wid == 0)
    def _():
        pltpu.sync_copy(pos_hbm, pos_v)

        def zero_step(i, _):
            src_v[pl.ds(i * 16, 16)] = jnp.zeros((16,), jnp.int32)
            return 0

        lax.fori_loop(0, NP_PAD // 16, zero_step, 0)

        def scat_step(i, _):
            pv = pos_v[pl.ds(i * 16, 16)]
            vals = lax.broadcasted_iota(jnp.int32, (16,), 0) + i * 16
            plsc.store_scatter(src_v, [pv], vals)
            return 0

        lax.fori_loop(0, (TOKENS * TOP_K) // 16, scat_step, 0)
        pltpu.sync_copy(src_v, src_hbm)

    del zero16


GC = 24


def _gather_body(src_hbm, x_hbm, xs_hbm, idx0, idx1, rows0, rows1,
                 sem0, sem1):
    wid = lax.axis_index("s") * 2 + lax.axis_index("c")
    base = wid * ROWS_PER_W
    n_chunks = ROWS_PER_W // GC
    idx = (idx0, idx1)
    rows = (rows0, rows1)
    sems = (sem0, sem1)

    def start(j):
        b = j % 2
        pltpu.sync_copy(src_hbm.at[pl.ds(base + j * GC, GC)], idx[b])
        pltpu.async_copy(x_hbm.at[idx[b]], rows[b], sems[b])

    start(0)
    for j in range(n_chunks):
        if j + 1 < n_chunks:
            start(j + 1)
        b = j % 2
        pltpu.make_async_copy(x_hbm.at[idx[b]], rows[b], sems[b]).wait()
        pltpu.sync_copy(rows[b], xs_hbm.at[pl.ds(base + j * GC, GC)])


IB = 4
IQ = INTER // IB


def _grouped_body(blk_ref, x_ref, wg_ref, wu_ref, w2_ref, y_ref, xb_ref):
    del blk_ref
    i = pl.program_id(0)
    q = pl.program_id(1)
    qe = q + (i % 2) * (IB - 1 - 2 * q)

    @pl.when(q == 0)
    def _():
        xb_ref[...] = x_ref[...].astype(jnp.bfloat16)

    xb = xb_ref[...]
    wg = wg_ref[0].astype(jnp.bfloat16)
    wu = wu_ref[0].astype(jnp.bfloat16)
    gate = lax.dot_general(
        xb, wg, (((1,), (1,)), ((), ())),
        preferred_element_type=jnp.float32)
    up = lax.dot_general(
        xb, wu, (((1,), (1,)), ((), ())),
        preferred_element_type=jnp.float32)
    act = (gate * lax.logistic(gate) * up).astype(jnp.bfloat16)
    for k in range(IB):
        @pl.when((qe == k) & (q == 0))
        def _(k=k):
            w2k = w2_ref[0, :, pl.ds(k * IQ, IQ)]
            y_ref[...] = lax.dot_general(
                act, w2k, (((1,), (1,)), ((), ())),
                preferred_element_type=jnp.float32)

        @pl.when((qe == k) & (q > 0))
        def _(k=k):
            w2k = w2_ref[0, :, pl.ds(k * IQ, IQ)]
            y_ref[...] += lax.dot_general(
                act, w2k, (((1,), (1,)), ((), ())),
                preferred_element_type=jnp.float32)


def _shared_body(x_ref, wg_ref, wu_ref, wd_ref, out_ref, xb_ref):
    h = pl.program_id(1)

    @pl.when(h == 0)
    def _():
        xb_ref[...] = x_ref[...].astype(jnp.bfloat16)

    xb = xb_ref[...]
    wg = wg_ref[0]
    wu = wu_ref[0]
    gate = lax.dot_general(
        xb, wg, (((1,), (1,)), ((), ())),
        preferred_element_type=jnp.float32)
    up = lax.dot_general(
        xb, wu, (((1,), (1,)), ((), ())),
        preferred_element_type=jnp.float32)
    act = (gate * lax.logistic(gate) * up).astype(jnp.bfloat16)
    wd = wd_ref[...]
    o = lax.dot_general(
        act, wd, (((1,), (1,)), ((), ())),
        preferred_element_type=jnp.float32)

    @pl.when(h == 0)
    def _():
        out_ref[...] = o

    @pl.when(h > 0)
    def _():
        out_ref[...] += o


def _cast_body(src_ref, dst_ref):
    dst_ref[...] = src_ref[...].astype(jnp.bfloat16)


def _combine_body(y_hbm, pos_hbm, w_hbm, out_hbm,
                  idx0, idx1, w_v, yb0, yb1, ob_v, sem0, sem1):
    wid = lax.axis_index("s") * 2 + lax.axis_index("c")
    tok0 = wid * TOK_PER_W
    pair0 = wid * TOK_PER_W * 2
    n_chunks = TOK_PER_W // 8
    pltpu.sync_copy(w_hbm.at[pl.ds(pair0, TOK_PER_W * 2)], w_v)
    idx = (idx0, idx1)
    yb = (yb0, yb1)
    sems = (sem0, sem1)

    def start(c, b):
        pltpu.sync_copy(pos_hbm.at[pl.ds(pair0 + c * 16, 16)], idx[b])
        pltpu.async_copy(y_hbm.at[idx[b]], yb[b], sems[b])

    def compute(c, b):
        pltpu.make_async_copy(y_hbm.at[idx[b]], yb[b], sems[b]).wait()
        for r in range(8):
            w0 = plsc.load_gather(
                w_v, [jnp.full((16,), c * 16 + 2 * r, jnp.int32)])
            w1 = plsc.load_gather(
                w_v, [jnp.full((16,), c * 16 + 2 * r + 1, jnp.int32)])

            def dstep(d, _):
                ds = pl.ds(d * 16, 16)
                ob_v[r, ds] = (w0 * yb[b][2 * r, ds]
                               + w1 * yb[b][2 * r + 1, ds])
                return 0

            lax.fori_loop(0, HIDDEN // 16, dstep, 0)
        pltpu.sync_copy(ob_v, out_hbm.at[pl.ds(tok0 + c * 8, 8)])

    start(0, 0)

    def c2step(c2, _):
        a = 2 * c2
        start(a + 1, 1)
        compute(a, 0)

        @pl.when(c2 < n_chunks // 2 - 1)
        def _():
            start(a + 2, 0)

        compute(a + 1, 1)
        return 0

    lax.fori_loop(0, n_chunks // 2, c2step, 0)


@jax.jit
def kernel(hidden_states, router_weight, w13, w2, w_gate, w_up, w_down):
    T, H = hidden_states.shape
    E = NUM_EXPERTS
    I = INTER
    n_tb = T // TB

    topk_idx, topk_w = pl.pallas_call(
        _router_body,
        out_shape=(jax.ShapeDtypeStruct((T, 2), jnp.int32),
                   jax.ShapeDtypeStruct((T, 2), jnp.float32)),
        grid=(n_tb,),
        in_specs=[
            pl.BlockSpec((TB, H), lambda t: (t, 0)),
            pl.BlockSpec((E, H), lambda t: (0, 0)),
        ],
        out_specs=(pl.BlockSpec((TB, 2), lambda t: (t, 0)),
                   pl.BlockSpec((TB, 2), lambda t: (t, 0))),
    )(hidden_states, router_weight)

    flat_e = topk_idx.reshape(64, 128)
    pos64, blk = pl.pallas_call(
        _rank_body,
        out_shape=(jax.ShapeDtypeStruct((64, 128), jnp.int32),
                   jax.ShapeDtypeStruct((1, 128), jnp.int32)),
    )(flat_e)
    pos = pos64.reshape(T * 2)
    blk_s = blk.reshape(128)

    src = pl.kernel(
        _scatter_body,
        out_type=jax.ShapeDtypeStruct((NP_PAD,), jnp.int32),
        mesh=plsc.VectorSubcoreMesh(core_axis_name="c", subcore_axis_name="s"),
        compiler_params=pltpu.CompilerParams(needs_layout_passes=False),
        scratch_types=[
            pltpu.VMEM((T * 2,), jnp.int32),
            pltpu.VMEM((NP_PAD,), jnp.int32),
            pltpu.VMEM((16,), jnp.int32),
        ],
    )(pos)

    x_sorted = pl.kernel(
        _gather_body,
        out_type=jax.ShapeDtypeStruct((NP_PAD, H), jnp.float32),
        mesh=plsc.VectorSubcoreMesh(core_axis_name="c", subcore_axis_name="s"),
        compiler_params=pltpu.CompilerParams(needs_layout_passes=False),
        scratch_types=[
            pltpu.VMEM((GC,), jnp.int32),
            pltpu.VMEM((GC,), jnp.int32),
            pltpu.VMEM((GC, H), jnp.float32),
            pltpu.VMEM((GC, H), jnp.float32),
            pltpu.SemaphoreType.DMA,
            pltpu.SemaphoreType.DMA,
        ],
    )(src, hidden_states)

    w2b = pl.pallas_call(
        _cast_body,
        out_shape=jax.ShapeDtypeStruct((E, H, I), jnp.bfloat16),
        grid=(E,),
        in_specs=[pl.BlockSpec((1, H, I), lambda i: (i, 0, 0))],
        out_specs=pl.BlockSpec((1, H, I), lambda i: (i, 0, 0)),
    )(w2)

    def _zig(i, q):
        return q + (i % 2) * (IB - 1 - 2 * q)

    y = pl.pallas_call(
        _grouped_body,
        out_shape=jax.ShapeDtypeStruct((NP_PAD, H), jnp.float32),
        grid_spec=pltpu.PrefetchScalarGridSpec(
            num_scalar_prefetch=1,
            grid=(NB, IB),
            in_specs=[
                pl.BlockSpec((B, H), lambda i, q, b: (i, 0)),
                pl.BlockSpec((1, IQ, H),
                             lambda i, q, b: (b[i], _zig(i, q), 0)),
                pl.BlockSpec((1, IQ, H),
                             lambda i, q, b: (b[i], IB + _zig(i, q), 0)),
                pl.BlockSpec((1, H, I), lambda i, q, b: (b[i], 0, 0)),
            ],
            out_specs=pl.BlockSpec((B, H), lambda i, q, b: (i, 0)),
            scratch_shapes=[pltpu.VMEM((B, H), jnp.bfloat16)],
        ),
        compiler_params=pltpu.CompilerParams(
            dimension_semantics=("arbitrary", "arbitrary")),
    )(blk_s, x_sorted, w13, w13, w2b)

    def _cast_2d(w):
        return pl.pallas_call(
            _cast_body,
            out_shape=jax.ShapeDtypeStruct((2 * I, H), jnp.bfloat16),
            grid=(4,),
            in_specs=[pl.BlockSpec((I // 2, H), lambda i: (i, 0))],
            out_specs=pl.BlockSpec((I // 2, H), lambda i: (i, 0)),
        )(w)

    wgb = _cast_2d(w_gate).reshape(2, I, H)
    wub = _cast_2d(w_up).reshape(2, I, H)
    wdb = w_down.astype(jnp.bfloat16)

    shared_out = pl.pallas_call(
        _shared_body,
        out_shape=jax.ShapeDtypeStruct((T, H), jnp.float32),
        grid=(n_tb, 2),
        in_specs=[
            pl.BlockSpec((TB, H), lambda t, h: (t, 0)),
            pl.BlockSpec((1, I, H), lambda t, h: (h, 0, 0)),
            pl.BlockSpec((1, I, H), lambda t, h: (h, 0, 0)),
            pl.BlockSpec((H, I), lambda t, h: (0, h)),
        ],
        out_specs=pl.BlockSpec((TB, H), lambda t, h: (t, 0)),
        scratch_shapes=[pltpu.VMEM((TB, H), jnp.bfloat16)],
        compiler_params=pltpu.CompilerParams(
            dimension_semantics=("parallel", "arbitrary")),
    )(hidden_states, wgb, wub, wdb)

    sparse_out = pl.kernel(
        _combine_body,
        out_type=jax.ShapeDtypeStruct((T, H), jnp.float32),
        mesh=plsc.VectorSubcoreMesh(core_axis_name="c", subcore_axis_name="s"),
        compiler_params=pltpu.CompilerParams(needs_layout_passes=False),
        scratch_types=[
            pltpu.VMEM((16,), jnp.int32),
            pltpu.VMEM((16,), jnp.int32),
            pltpu.VMEM((TOK_PER_W * 2,), jnp.float32),
            pltpu.VMEM((16, H), jnp.float32),
            pltpu.VMEM((16, H), jnp.float32),
            pltpu.VMEM((8, H), jnp.float32),
            pltpu.SemaphoreType.DMA,
            pltpu.SemaphoreType.DMA,
        ],
    )(y, pos, topk_w.reshape(T * 2))

    return sparse_out + shared_out


# speedup vs baseline: 1.4597x; 1.0199x over previous
import functools

import jax
import jax.numpy as jnp
from jax import lax
from jax.experimental import pallas as pl
from jax.experimental.pallas import tpu as pltpu
from jax.experimental.pallas import tpu_sc as plsc

NUM_EXPERTS = 8
TOP_K = 2
HIDDEN = 2048
INTER = 1792
TOKENS = 4096

TB = 256
B = 256
NB = 32 + 7
NP = NB * B
NP_PAD = NP
NW = 32
ROWS_PER_W = NP_PAD // NW
TOK_PER_W = TOKENS // NW


def _router_body(x_ref, rw_ref, idx_ref, w_ref):
    x = x_ref[...].astype(jnp.bfloat16)
    rw = rw_ref[...].astype(jnp.bfloat16)
    logits = lax.dot_general(
        x, rw, (((1,), (1,)), ((), ())),
        preferred_element_type=jnp.float32)
    e_iota = lax.broadcasted_iota(jnp.int32, logits.shape, 1)
    m1 = jnp.max(logits, axis=1, keepdims=True)
    i1 = jnp.min(jnp.where(logits == m1, e_iota, NUM_EXPERTS), axis=1,
                 keepdims=True)
    masked = jnp.where(e_iota == i1, -jnp.inf, logits)
    m2 = jnp.max(masked, axis=1, keepdims=True)
    i2 = jnp.min(jnp.where(masked == m2, e_iota, NUM_EXPERTS), axis=1,
                 keepdims=True)
    b = jnp.exp(m2 - m1)
    denom = 1.0 + b
    idx_ref[...] = jnp.concatenate([i1, i2], axis=1)
    w_ref[...] = jnp.concatenate([1.0 / denom, b / denom], axis=1)


def _rank_body(e_ref, pos_ref, blk_ref):
    a = e_ref[...]
    rows, cols = a.shape
    r1 = lax.broadcasted_iota(jnp.int32, (rows, rows), 0)
    c1 = lax.broadcasted_iota(jnp.int32, (rows, rows), 1)
    l_strict = jnp.where(r1 > c1, 1.0, 0.0).astype(jnp.bfloat16)
    r2 = lax.broadcasted_iota(jnp.int32, (cols, cols), 0)
    c2 = lax.broadcasted_iota(jnp.int32, (cols, cols), 1)
    u_strict = jnp.where(r2 < c2, 1.0, 0.0).astype(jnp.bfloat16)

    j_f = lax.broadcasted_iota(jnp.int32, (1, cols), 1).astype(jnp.float32)
    pos_acc = jnp.zeros(a.shape, jnp.float32)
    blk_acc = jnp.zeros((1, cols), jnp.float32)
    g = jnp.float32(0.0)
    cum_nb = jnp.float32(0.0)
    for e in range(NUM_EXPERTS):
        m = (a == e).astype(jnp.float32)
        mb = m.astype(jnp.bfloat16)
        w_in_row = lax.dot_general(
            mb, u_strict, (((1,), (0,)), ((), ())),
            preferred_element_type=jnp.float32)
        q = lax.dot_general(
            l_strict, mb, (((1,), (0,)), ((), ())),
            preferred_element_type=jnp.float32)
        prefix_row = jnp.sum(q, axis=1, keepdims=True)
        rank = prefix_row + w_in_row
        c_e = jnp.sum(m)
        nb_e = jnp.floor((c_e + (B - 1.0)) / B)
        pos_acc = pos_acc + m * (g + rank)
        g = g + nb_e * B
        cum_nb = cum_nb + nb_e
        blk_acc = blk_acc + jnp.where(j_f >= cum_nb, 1.0, 0.0)
    pos_ref[...] = pos_acc.astype(jnp.int32)
    blk_ref[...] = jnp.minimum(blk_acc, NUM_EXPERTS - 1.0).astype(jnp.int32)


def _scatter_body(pos_hbm, src_hbm, pos_v, src_v, zero16):
    wid = lax.axis_index("s") * 2 + lax.axis_index("c")

    @pl.when(wid == 0)
    def _():
        pltpu.sync_copy(pos_hbm, pos_v)

        def zero_step(i, _):
            src_v[pl.ds(i * 16, 16)] = jnp.zeros((16,), jnp.int32)
            return 0

        lax.fori_loop(0, NP_PAD // 16, zero_step, 0)

        def scat_step(i, _):
            pv = pos_v[pl.ds(i * 16, 16)]
            vals = lax.broadcasted_iota(jnp.int32, (16,), 0) + i * 16
            plsc.store_scatter(src_v, [pv], vals)
            return 0

        lax.fori_loop(0, (TOKENS * TOP_K) // 16, scat_step, 0)
        pltpu.sync_copy(src_v, src_hbm)

    del zero16


GC = 24


def _gather_body(src_hbm, x_hbm, xs_hbm, idx0, idx1, rows0, rows1,
                 sem0, sem1):
    wid = lax.axis_index("s") * 2 + lax.axis_index("c")
    base = wid * ROWS_PER_W
    n_chunks = ROWS_PER_W // GC
    idx = (idx0, idx1)
    rows = (rows0, rows1)
    sems = (sem0, sem1)

    def start(j):
        b = j % 2
        pltpu.sync_copy(src_hbm.at[pl.ds(base + j * GC, GC)], idx[b])
        pltpu.async_copy(x_hbm.at[idx[b]], rows[b], sems[b])

    start(0)
    for j in range(n_chunks):
        if j + 1 < n_chunks:
            start(j + 1)
        b = j % 2
        pltpu.make_async_copy(x_hbm.at[idx[b]], rows[b], sems[b]).wait()
        pltpu.sync_copy(rows[b], xs_hbm.at[pl.ds(base + j * GC, GC)])


IB = 4
IQ = INTER // IB


def _grouped_body(blk_ref, x_ref, wg_ref, wu_ref, w2_ref, y_ref, xb_ref):
    del blk_ref
    i = pl.program_id(0)
    q = pl.program_id(1)
    qe = q + (i % 2) * (IB - 1 - 2 * q)

    @pl.when(q == 0)
    def _():
        xb_ref[...] = x_ref[...].astype(jnp.bfloat16)

    xb = xb_ref[...]
    wg = wg_ref[0].astype(jnp.bfloat16)
    wu = wu_ref[0].astype(jnp.bfloat16)
    gate = lax.dot_general(
        xb, wg, (((1,), (1,)), ((), ())),
        preferred_element_type=jnp.float32)
    up = lax.dot_general(
        xb, wu, (((1,), (1,)), ((), ())),
        preferred_element_type=jnp.float32)
    act = (gate * lax.logistic(gate) * up).astype(jnp.bfloat16)
    for k in range(IB):
        @pl.when((qe == k) & (q == 0))
        def _(k=k):
            w2k = w2_ref[0, :, pl.ds(k * IQ, IQ)].astype(jnp.bfloat16)
            y_ref[...] = lax.dot_general(
                act, w2k, (((1,), (1,)), ((), ())),
                preferred_element_type=jnp.float32)

        @pl.when((qe == k) & (q > 0))
        def _(k=k):
            w2k = w2_ref[0, :, pl.ds(k * IQ, IQ)].astype(jnp.bfloat16)
            y_ref[...] += lax.dot_general(
                act, w2k, (((1,), (1,)), ((), ())),
                preferred_element_type=jnp.float32)


def _shared_body(x_ref, wg_ref, wu_ref, wd_ref, out_ref, xb_ref):
    h = pl.program_id(1)

    @pl.when(h == 0)
    def _():
        xb_ref[...] = x_ref[...].astype(jnp.bfloat16)

    xb = xb_ref[...]
    wg = wg_ref[0]
    wu = wu_ref[0]
    gate = lax.dot_general(
        xb, wg, (((1,), (1,)), ((), ())),
        preferred_element_type=jnp.float32)
    up = lax.dot_general(
        xb, wu, (((1,), (1,)), ((), ())),
        preferred_element_type=jnp.float32)
    act = (gate * lax.logistic(gate) * up).astype(jnp.bfloat16)
    wd = wd_ref[...]
    o = lax.dot_general(
        act, wd, (((1,), (1,)), ((), ())),
        preferred_element_type=jnp.float32)

    @pl.when(h == 0)
    def _():
        out_ref[...] = o

    @pl.when(h > 0)
    def _():
        out_ref[...] += o


def _cast_body(src_ref, dst_ref):
    dst_ref[...] = src_ref[...].astype(jnp.bfloat16)


def _combine_body(y_hbm, pos_hbm, w_hbm, out_hbm,
                  idx0, idx1, w_v, yb0, yb1, ob_v, sem0, sem1):
    wid = lax.axis_index("s") * 2 + lax.axis_index("c")
    tok0 = wid * TOK_PER_W
    pair0 = wid * TOK_PER_W * 2
    n_chunks = TOK_PER_W // 8
    pltpu.sync_copy(w_hbm.at[pl.ds(pair0, TOK_PER_W * 2)], w_v)
    idx = (idx0, idx1)
    yb = (yb0, yb1)
    sems = (sem0, sem1)

    def start(c, b):
        pltpu.sync_copy(pos_hbm.at[pl.ds(pair0 + c * 16, 16)], idx[b])
        pltpu.async_copy(y_hbm.at[idx[b]], yb[b], sems[b])

    def compute(c, b):
        pltpu.make_async_copy(y_hbm.at[idx[b]], yb[b], sems[b]).wait()
        for r in range(8):
            w0 = plsc.load_gather(
                w_v, [jnp.full((16,), c * 16 + 2 * r, jnp.int32)])
            w1 = plsc.load_gather(
                w_v, [jnp.full((16,), c * 16 + 2 * r + 1, jnp.int32)])

            def dstep(d, _):
                ds = pl.ds(d * 16, 16)
                ob_v[r, ds] = (w0 * yb[b][2 * r, ds]
                               + w1 * yb[b][2 * r + 1, ds])
                return 0

            lax.fori_loop(0, HIDDEN // 16, dstep, 0)
        pltpu.sync_copy(ob_v, out_hbm.at[pl.ds(tok0 + c * 8, 8)])

    start(0, 0)

    def c2step(c2, _):
        a = 2 * c2
        start(a + 1, 1)
        compute(a, 0)

        @pl.when(c2 < n_chunks // 2 - 1)
        def _():
            start(a + 2, 0)

        compute(a + 1, 1)
        return 0

    lax.fori_loop(0, n_chunks // 2, c2step, 0)


@jax.jit
def kernel(hidden_states, router_weight, w13, w2, w_gate, w_up, w_down):
    T, H = hidden_states.shape
    E = NUM_EXPERTS
    I = INTER
    n_tb = T // TB

    topk_idx, topk_w = pl.pallas_call(
        _router_body,
        out_shape=(jax.ShapeDtypeStruct((T, 2), jnp.int32),
                   jax.ShapeDtypeStruct((T, 2), jnp.float32)),
        grid=(n_tb,),
        in_specs=[
            pl.BlockSpec((TB, H), lambda t: (t, 0)),
            pl.BlockSpec((E, H), lambda t: (0, 0)),
        ],
        out_specs=(pl.BlockSpec((TB, 2), lambda t: (t, 0)),
                   pl.BlockSpec((TB, 2), lambda t: (t, 0))),
    )(hidden_states, router_weight)

    flat_e = topk_idx.reshape(64, 128)
    pos64, blk = pl.pallas_call(
        _rank_body,
        out_shape=(jax.ShapeDtypeStruct((64, 128), jnp.int32),
                   jax.ShapeDtypeStruct((1, 128), jnp.int32)),
    )(flat_e)
    pos = pos64.reshape(T * 2)
    blk_s = blk.reshape(128)

    src = pl.kernel(
        _scatter_body,
        out_type=jax.ShapeDtypeStruct((NP_PAD,), jnp.int32),
        mesh=plsc.VectorSubcoreMesh(core_axis_name="c", subcore_axis_name="s"),
        compiler_params=pltpu.CompilerParams(needs_layout_passes=False),
        scratch_types=[
            pltpu.VMEM((T * 2,), jnp.int32),
            pltpu.VMEM((NP_PAD,), jnp.int32),
            pltpu.VMEM((16,), jnp.int32),
        ],
    )(pos)

    x_sorted = pl.kernel(
        _gather_body,
        out_type=jax.ShapeDtypeStruct((NP_PAD, H), jnp.float32),
        mesh=plsc.VectorSubcoreMesh(core_axis_name="c", subcore_axis_name="s"),
        compiler_params=pltpu.CompilerParams(needs_layout_passes=False),
        scratch_types=[
            pltpu.VMEM((GC,), jnp.int32),
            pltpu.VMEM((GC,), jnp.int32),
            pltpu.VMEM((GC, H), jnp.float32),
            pltpu.VMEM((GC, H), jnp.float32),
            pltpu.SemaphoreType.DMA,
            pltpu.SemaphoreType.DMA,
        ],
    )(src, hidden_states)

    def _zig(i, q):
        return q + (i % 2) * (IB - 1 - 2 * q)

    y = pl.pallas_call(
        _grouped_body,
        out_shape=jax.ShapeDtypeStruct((NP_PAD, H), jnp.float32),
        grid_spec=pltpu.PrefetchScalarGridSpec(
            num_scalar_prefetch=1,
            grid=(NB, IB),
            in_specs=[
                pl.BlockSpec((B, H), lambda i, q, b: (i, 0)),
                pl.BlockSpec((1, IQ, H),
                             lambda i, q, b: (b[i], _zig(i, q), 0)),
                pl.BlockSpec((1, IQ, H),
                             lambda i, q, b: (b[i], IB + _zig(i, q), 0)),
                pl.BlockSpec((1, H, I), lambda i, q, b: (b[i], 0, 0)),
            ],
            out_specs=pl.BlockSpec((B, H), lambda i, q, b: (i, 0)),
            scratch_shapes=[pltpu.VMEM((B, H), jnp.bfloat16)],
        ),
        compiler_params=pltpu.CompilerParams(
            dimension_semantics=("arbitrary", "arbitrary")),
    )(blk_s, x_sorted, w13, w13, w2)

    def _cast_2d(w):
        return pl.pallas_call(
            _cast_body,
            out_shape=jax.ShapeDtypeStruct((2 * I, H), jnp.bfloat16),
            grid=(4,),
            in_specs=[pl.BlockSpec((I // 2, H), lambda i: (i, 0))],
            out_specs=pl.BlockSpec((I // 2, H), lambda i: (i, 0)),
        )(w)

    wgb = _cast_2d(w_gate).reshape(2, I, H)
    wub = _cast_2d(w_up).reshape(2, I, H)
    wdb = w_down.astype(jnp.bfloat16)

    shared_out = pl.pallas_call(
        _shared_body,
        out_shape=jax.ShapeDtypeStruct((T, H), jnp.float32),
        grid=(n_tb, 2),
        in_specs=[
            pl.BlockSpec((TB, H), lambda t, h: (t, 0)),
            pl.BlockSpec((1, I, H), lambda t, h: (h, 0, 0)),
            pl.BlockSpec((1, I, H), lambda t, h: (h, 0, 0)),
            pl.BlockSpec((H, I), lambda t, h: (0, h)),
        ],
        out_specs=pl.BlockSpec((TB, H), lambda t, h: (t, 0)),
        scratch_shapes=[pltpu.VMEM((TB, H), jnp.bfloat16)],
        compiler_params=pltpu.CompilerParams(
            dimension_semantics=("parallel", "arbitrary")),
    )(hidden_states, wgb, wub, wdb)

    sparse_out = pl.kernel(
        _combine_body,
        out_type=jax.ShapeDtypeStruct((T, H), jnp.float32),
        mesh=plsc.VectorSubcoreMesh(core_axis_name="c", subcore_axis_name="s"),
        compiler_params=pltpu.CompilerParams(needs_layout_passes=False),
        scratch_types=[
            pltpu.VMEM((16,), jnp.int32),
            pltpu.VMEM((16,), jnp.int32),
            pltpu.VMEM((TOK_PER_W * 2,), jnp.float32),
            pltpu.VMEM((16, H), jnp.float32),
            pltpu.VMEM((16, H), jnp.float32),
            pltpu.VMEM((8, H), jnp.float32),
            pltpu.SemaphoreType.DMA,
            pltpu.SemaphoreType.DMA,
        ],
    )(y, pos, topk_w.reshape(T * 2))

    return sparse_out + shared_out
